# baseline (speedup 1.0000x reference)
import functools

import jax
import jax.numpy as jnp
from jax import lax
from jax.experimental import pallas as pl
from jax.experimental.pallas import tpu as pltpu
from jax.experimental.pallas import tpu_sc as plsc

_NC = 2
_NS = 16
_NW = _NC * _NS
_D = 128

_C = 13312
_CPT = _C // _NS
_CAP = 128
_BS = 2048
_COMP = _CAP + 16 + 16


def _spmm_sc(rows, cols, vals, Y, n_out):
    E = rows.shape[0]
    n_in = Y.shape[0]
    ep_unit = _NW * _BS
    Ep = ((E + ep_unit - 1) // ep_unit) * ep_unit
    if Ep != E:
        pad = Ep - E
        z = jnp.zeros((pad,), jnp.int32)
        rows = jnp.concatenate([rows.astype(jnp.int32), z])
        cols = jnp.concatenate([cols.astype(jnp.int32), z])
        vals = jnp.concatenate([vals, jnp.zeros((pad,), vals.dtype)])
    else:
        rows = rows.astype(jnp.int32)
        cols = cols.astype(jnp.int32)
    chunks = (n_out + _C - 1) // _C
    n_pad = chunks * _C
    Et = Ep // _NW
    NB = Et // _BS

    mesh = plsc.VectorSubcoreMesh(core_axis_name="c", subcore_axis_name="s")

    @functools.partial(
        pl.kernel,
        mesh=mesh,
        compiler_params=pltpu.CompilerParams(needs_layout_passes=False),
        out_type=jax.ShapeDtypeStruct((2, n_pad, _D), jnp.float32),
        scratch_types=[
            pltpu.VMEM_SHARED((_C + 8, _D), jnp.float32),
            pltpu.VMEM((_BS,), jnp.int32),
            pltpu.VMEM((_BS,), jnp.int32),
            pltpu.VMEM((_BS,), jnp.float32),
            pltpu.VMEM((_COMP,), jnp.int32),
            pltpu.VMEM((_COMP,), jnp.int32),
            pltpu.VMEM((_COMP,), jnp.float32),
            pltpu.VMEM((_CAP,), jnp.int32),
            pltpu.VMEM((_CAP,), jnp.int32),
            pltpu.VMEM((_CAP, _D), jnp.float32),
        ],
    )
    def k(rows_h, cols_h, vals_h, y_h, z_h, out_h,
          acc, rblk, cblk, vblk, ccol, cloc, cval, fcol, floc, grows):
        c = lax.axis_index("c")
        s = lax.axis_index("s")
        wid = s * _NC + c
        base_e = wid * Et
        zero16 = jnp.zeros((16,), jnp.float32)
        zero16i = jnp.zeros((16,), jnp.int32)
        lane = lax.iota(jnp.int32, 16)

        for q in range(_COMP // 16):
            ccol[pl.ds(q * 16, 16)] = zero16i
            cloc[pl.ds(q * 16, 16)] = zero16i
            cval[pl.ds(q * 16, 16)] = zero16

        def flush():
            for q in range(_CAP // 16):
                fcol[pl.ds(q * 16, 16)] = ccol[pl.ds(q * 16, 16)]
                floc[pl.ds(q * 16, 16)] = cloc[pl.ds(q * 16, 16)]
            pltpu.sync_copy(y_h.at[fcol], grows)
            def _scale(q, _):
                vv = cval[pl.ds(q * 16, 16)]
                for l in range(16):
                    sv = jnp.sum(jnp.where(lane == l, vv, 0.0))
                    r = q * 16 + l
                    for g in range(8):
                        grows[r, pl.ds(g * 16, 16)] = (
                            grows[r, pl.ds(g * 16, 16)] * sv)
                return 0
            lax.fori_loop(0, _CAP // 16, _scale, 0)
            pltpu.sync_copy(grows, acc.at[floc], add=True)
            ccol[pl.ds(0, 16)] = ccol[pl.ds(_CAP, 16)]
            cloc[pl.ds(0, 16)] = cloc[pl.ds(_CAP, 16)]
            cval[pl.ds(0, 16)] = cval[pl.ds(_CAP, 16)]

        def chunk_body(ch, _):
            r0 = ch * _C
            for j in range(6):
                pltpu.sync_copy(z_h, acc.at[pl.ds(s * _CPT + j * 128, 128)])
            pltpu.sync_copy(z_h.at[pl.ds(0, 64)],
                            acc.at[pl.ds(s * _CPT + 768, 64)])
            plsc.subcore_barrier()

            def block_body(b, off):
                eb = base_e + b * _BS
                pltpu.sync_copy(rows_h.at[pl.ds(eb, _BS)], rblk)
                pltpu.sync_copy(cols_h.at[pl.ds(eb, _BS)], cblk)
                pltpu.sync_copy(vals_h.at[pl.ds(eb, _BS)], vblk)

                def group_body(g, off):
                    r = rblk[pl.ds(g * 16, 16)]
                    cc = cblk[pl.ds(g * 16, 16)]
                    vv = vblk[pl.ds(g * 16, 16)]
                    m = jnp.logical_and(r >= r0, r < r0 + _C)
                    loc = jnp.where(m, r - r0, _C)
                    pcs = plsc.cumsum(jnp.where(m, 1, 0))
                    dst = jnp.where(m, off + pcs - 1, _CAP + 16)
                    plsc.store_scatter(ccol, [dst], cc)
                    plsc.store_scatter(cloc, [dst], loc)
                    plsc.store_scatter(cval, [dst], vv)
                    off = off + jnp.max(pcs)
                    do_flush = off >= _CAP
                    pl.when(do_flush)(flush)
                    return jnp.where(do_flush, off - _CAP, off)

                return lax.fori_loop(0, _BS // 16, group_body, off)

            off = lax.fori_loop(0, NB, block_body, jnp.int32(0))

            def tail():
                for q in range(_COMP // 16):
                    idx = lane + q * 16
                    keep = idx < off
                    lq = cloc[pl.ds(q * 16, 16)]
                    vq = cval[pl.ds(q * 16, 16)]
                    cloc[pl.ds(q * 16, 16)] = jnp.where(keep, lq, _C)
                    cval[pl.ds(q * 16, 16)] = jnp.where(keep, vq, 0.0)
                flush()
            pl.when(off > 0)(tail)

            plsc.subcore_barrier()
            pltpu.sync_copy(
                acc.at[pl.ds(s * _CPT, _CPT)],
                out_h.at[c].at[pl.ds(r0 + s * _CPT, _CPT)])
            plsc.subcore_barrier()
            return 0

        lax.fori_loop(0, chunks, chunk_body, 0)

    return k(rows, cols, vals, Y, jnp.zeros((128, _D), jnp.float32))


_GB = 128


def _gather_product_sc(X0, idx_list, n_rows):
    nf = len(idx_list)
    per_w = ((n_rows + _NW * _GB - 1) // (_NW * _GB)) * _GB
    n_pad = per_w * _NW
    idxs = []
    for ix in idx_list:
        ix = ix.astype(jnp.int32)
        if n_pad != n_rows:
            ix = jnp.concatenate(
                [ix, jnp.zeros((n_pad - n_rows,), jnp.int32)])
        idxs.append(ix)
    nblk = per_w // _GB

    mesh = plsc.VectorSubcoreMesh(core_axis_name="c", subcore_axis_name="s")
    scratch = ([pltpu.VMEM((_GB,), jnp.int32) for _ in range(nf)]
               + [pltpu.VMEM((_GB, _D), jnp.float32) for _ in range(nf)])

    @functools.partial(
        pl.kernel,
        mesh=mesh,
        compiler_params=pltpu.CompilerParams(needs_layout_passes=False),
        out_type=jax.ShapeDtypeStruct((n_pad, _D), jnp.float32),
        scratch_types=scratch,
    )
    def k(x_h, *refs):
        idx_h = refs[:nf]
        out_h = refs[nf]
        ib = refs[nf + 1:nf + 1 + nf]
        rb = refs[nf + 1 + nf:]
        c = lax.axis_index("c")
        s = lax.axis_index("s")
        wid = s * _NC + c
        base = wid * per_w

        def body(b, _):
            o = base + b * _GB
            for f in range(nf):
                pltpu.sync_copy(idx_h[f].at[pl.ds(o, _GB)], ib[f])
            for f in range(nf):
                pltpu.sync_copy(x_h.at[ib[f]], rb[f])

            def mrow(i, _):
                for g in range(8):
                    sl = pl.ds(g * 16, 16)
                    m = rb[0][i, sl] != 0.0
                    for f in range(1, nf):
                        m = jnp.logical_and(m, rb[f][i, sl] != 0.0)
                    rb[0][i, sl] = jnp.where(m, 1.0, 0.0)
                return 0
            lax.fori_loop(0, _GB, mrow, 0)
            pltpu.sync_copy(rb[0], out_h.at[pl.ds(o, _GB)])
            return 0
        lax.fori_loop(0, nblk, body, 0)

    return k(X0, *idxs)


def _mm_bin_body(x_ref, w_ref, b_ref, o_ref):
    xb = jnp.where(x_ref[...] != 0, 1.0, 0.0)
    o_ref[...] = (
        jnp.dot(xb, w_ref[...], preferred_element_type=jnp.float32)
        + b_ref[...])


def _mm_bin(X, W, b, block=512):
    N, K = X.shape
    F = W.shape[1]
    return pl.pallas_call(
        _mm_bin_body,
        grid=(pl.cdiv(N, block),),
        in_specs=[
            pl.BlockSpec((block, K), lambda i: (i, 0)),
            pl.BlockSpec((K, F), lambda i: (0, 0)),
            pl.BlockSpec((1, F), lambda i: (0, 0)),
        ],
        out_specs=pl.BlockSpec((block, F), lambda i: (i, 0)),
        out_shape=jax.ShapeDtypeStruct((N, F), jnp.float32),
    )(X, W, b.reshape(1, F))


def _mm3_body(x_ref, w_ref, b_ref, o_ref):
    o_ref[0] = (
        jnp.dot(x_ref[...], w_ref[0], preferred_element_type=jnp.float32)
        + b_ref[0])


def _mm3(X, W3, b3, block=512):
    N, K = X.shape
    return pl.pallas_call(
        _mm3_body,
        grid=(3, pl.cdiv(N, block)),
        in_specs=[
            pl.BlockSpec((block, K), lambda g, i: (i, 0)),
            pl.BlockSpec((1, K, _D), lambda g, i: (g, 0, 0)),
            pl.BlockSpec((1, 1, _D), lambda g, i: (g, 0, 0)),
        ],
        out_specs=pl.BlockSpec((1, block, _D), lambda g, i: (g, i, 0)),
        out_shape=jax.ShapeDtypeStruct((3, N, _D), jnp.float32),
    )(X, W3, b3.reshape(3, 1, _D))


def _prelu2_body(a_ref, b_ref, w_ref, o_ref):
    h = a_ref[...] + b_ref[...]
    o_ref[...] = jnp.where(h >= 0, h, w_ref[0, 0] * h)


def _prelu_sum2(a, b, w, block=1024):
    N, F = a.shape
    return pl.pallas_call(
        _prelu2_body,
        grid=(pl.cdiv(N, block),),
        in_specs=[
            pl.BlockSpec((block, F), lambda i: (i, 0)),
            pl.BlockSpec((block, F), lambda i: (i, 0)),
            pl.BlockSpec((1, 1), lambda i: (0, 0)),
        ],
        out_specs=pl.BlockSpec((block, F), lambda i: (i, 0)),
        out_shape=jax.ShapeDtypeStruct((N, F), jnp.float32),
    )(a, b, w.reshape(1, 1))


def _tri_body(t0_ref, t1_ref, h0_ref, h1_ref, w_ref, b_ref, pw_ref, o_ref):
    h = h0_ref[...] + h1_ref[...]
    x1h = jnp.where(h >= 0, h, pw_ref[0, 0] * h)
    t = t0_ref[...] + t1_ref[...]
    o_ref[...] = x1h + (
        jnp.dot(t, w_ref[...], preferred_element_type=jnp.float32)
        + b_ref[...])


def _tri_merge(T0, T1, H0, H1, W, b, pw, block=512):
    N = T0.shape[0]
    return pl.pallas_call(
        _tri_body,
        grid=(pl.cdiv(N, block),),
        in_specs=[pl.BlockSpec((block, _D), lambda i: (i, 0))] * 4 + [
            pl.BlockSpec((_D, _D), lambda i: (0, 0)),
            pl.BlockSpec((1, _D), lambda i: (0, 0)),
            pl.BlockSpec((1, 1), lambda i: (0, 0)),
        ],
        out_specs=pl.BlockSpec((block, _D), lambda i: (i, 0)),
        out_shape=jax.ShapeDtypeStruct((N, _D), jnp.float32),
    )(T0, T1, H0, H1, W, b.reshape(1, _D), pw.reshape(1, 1))


def _final_body(h0_ref, h1_ref, s0_ref, s1_ref, pw_ref, o_ref):
    h = h0_ref[...] + h1_ref[...]
    x0h = jnp.where(h >= 0, h, pw_ref[0, 0] * h)
    o_ref[...] = (x0h + s0_ref[...] + s1_ref[...]) / 3.0


def _final(H0, H1, S0, S1, pw, block=1024):
    N = H0.shape[0]
    return pl.pallas_call(
        _final_body,
        grid=(pl.cdiv(N, block),),
        in_specs=[pl.BlockSpec((block, _D), lambda i: (i, 0))] * 4 + [
            pl.BlockSpec((1, 1), lambda i: (0, 0)),
        ],
        out_specs=pl.BlockSpec((block, _D), lambda i: (i, 0)),
        out_shape=jax.ShapeDtypeStruct((N, _D), jnp.float32),
    )(H0, H1, S0, S1, pw.reshape(1, 1))


def kernel(X0, X1_idx, X2_idx, L0_rows, L0_cols, L0_vals, L1_rows, L1_cols, L1_vals, L1u_rows, L1u_cols, L1u_vals, L1d_rows, L1d_cols, L1d_vals, L2_rows, L2_cols, L2_vals, B1_rows, B1_cols, B1_vals, B2_rows, B2_cols, B2_vals, Wn_u, bn_u, Wn_d, bn_d, Wn_p, bn_p, We_u, be_u, We_d, be_d, We_p, be_p, Wt_u, bt_u, Wt_d, bt_d, Wt_p, bt_p, W_tri, b_tri, prelu_w):
    N0 = X0.shape[0]
    N1 = X1_idx.shape[0]
    N2 = X2_idx.shape[0]

    Y0 = _mm_bin(X0, Wn_p + Wn_d, bn_p + bn_d)
    P0 = _spmm_sc(L0_rows, L0_cols, L0_vals, Y0, N0)
    H0a, H0b = P0[0, :N0], P0[1, :N0]

    X1f = _gather_product_sc(X0, [X1_idx[:, 0], X1_idx[:, 1]], N1)[:N1]
    W3 = jnp.stack([We_p, We_u, We_d])
    b3 = jnp.stack([be_p, be_u, be_d])
    Y1 = _mm3(X1f, W3, b3).reshape(3 * N1, _D)
    r1 = jnp.concatenate([L1_rows, L1u_rows, L1d_rows]).astype(jnp.int32)
    c1 = jnp.concatenate(
        [L1_cols.astype(jnp.int32),
         L1u_cols.astype(jnp.int32) + N1,
         L1d_cols.astype(jnp.int32) + 2 * N1])
    v1 = jnp.concatenate([L1_vals, L1u_vals, L1d_vals])
    P1 = _spmm_sc(r1, c1, v1, Y1, N1)
    H1a, H1b = P1[0, :N1], P1[1, :N1]

    X2f = _gather_product_sc(
        X0, [X2_idx[:, 0], X2_idx[:, 1], X2_idx[:, 2]], N2)[:N2]
    Y2 = _mm_bin(X2f, Wt_p + Wt_u, bt_p + bt_u)
    P2 = _spmm_sc(L2_rows, L2_cols, L2_vals, Y2, N2)
    X2h = _prelu_sum2(P2[0, :N2], P2[1, :N2], prelu_w)

    PT = _spmm_sc(B2_rows, B2_cols, B2_vals, X2h, N1)
    Sin = _tri_merge(PT[0, :N1], PT[1, :N1], H1a, H1b, W_tri, b_tri, prelu_w)
    PS = _spmm_sc(B1_rows, B1_cols, B1_vals, Sin, N0)
    return _final(H0a, H0b, PS[0, :N0], PS[1, :N0], prelu_w)


# speedup vs baseline: 1.1017x; 1.1017x over previous
import functools

import jax
import jax.numpy as jnp
from jax import lax
from jax.experimental import pallas as pl
from jax.experimental.pallas import tpu as pltpu
from jax.experimental.pallas import tpu_sc as plsc

_NC = 2
_NS = 16
_NW = _NC * _NS
_D = 128

_C = 12544
_CPT = _C // _NS
_CAP = 128
_BS = 2048
_WIN = _CAP + 64
_COMP = _WIN + 16


def _spmm_sc(rows, cols, vals, Y, n_out):
    E = rows.shape[0]
    ep_unit = _NW * _BS
    Ep = ((E + ep_unit - 1) // ep_unit) * ep_unit
    if Ep != E:
        pad = Ep - E
        z = jnp.zeros((pad,), jnp.int32)
        rows = jnp.concatenate([rows.astype(jnp.int32), z])
        cols = jnp.concatenate([cols.astype(jnp.int32), z])
        vals = jnp.concatenate([vals, jnp.zeros((pad,), vals.dtype)])
    else:
        rows = rows.astype(jnp.int32)
        cols = cols.astype(jnp.int32)
    chunks = (n_out + _C - 1) // _C
    n_pad = chunks * _C
    Et = Ep // _NW
    NB = Et // _BS

    mesh = plsc.VectorSubcoreMesh(core_axis_name="c", subcore_axis_name="s")

    @functools.partial(
        pl.kernel,
        mesh=mesh,
        compiler_params=pltpu.CompilerParams(needs_layout_passes=False),
        out_type=jax.ShapeDtypeStruct((2, n_pad, _D), jnp.float32),
        scratch_types=[
            pltpu.VMEM_SHARED((_C + 8, _D), jnp.float32),
            pltpu.VMEM((2, _BS), jnp.int32),
            pltpu.VMEM((2, _BS), jnp.int32),
            pltpu.VMEM((2, _BS), jnp.float32),
            pltpu.VMEM((_COMP,), jnp.int32),
            pltpu.VMEM((_COMP,), jnp.int32),
            pltpu.VMEM((_COMP,), jnp.float32),
            pltpu.VMEM((_CAP,), jnp.int32),
            pltpu.VMEM((_CAP,), jnp.int32),
            pltpu.VMEM((_CAP, _D), jnp.float32),
            pltpu.SemaphoreType.DMA,
        ],
    )
    def k(rows_h, cols_h, vals_h, y_h, z_h, out_h,
          acc, rblk, cblk, vblk, ccol, cloc, cval, fcol, floc, grows, sem):
        c = lax.axis_index("c")
        s = lax.axis_index("s")
        wid = s * _NC + c
        base_e = wid * Et
        zero16 = jnp.zeros((16,), jnp.float32)
        zero16i = jnp.zeros((16,), jnp.int32)
        lane = lax.iota(jnp.int32, 16)

        for q in range(_COMP // 16):
            ccol[pl.ds(q * 16, 16)] = zero16i
            cloc[pl.ds(q * 16, 16)] = zero16i
            cval[pl.ds(q * 16, 16)] = zero16

        def flush():
            for q in range(_CAP // 16):
                fcol[pl.ds(q * 16, 16)] = ccol[pl.ds(q * 16, 16)]
                floc[pl.ds(q * 16, 16)] = cloc[pl.ds(q * 16, 16)]
            pltpu.sync_copy(y_h.at[fcol], grows)
            def _scale(q, _):
                vv = cval[pl.ds(q * 16, 16)]
                for l in range(16):
                    sv = jnp.take(vv, jnp.full((16,), l, jnp.int32))
                    r = q * 16 + l
                    for g in range(8):
                        grows[r, pl.ds(g * 16, 16)] = (
                            grows[r, pl.ds(g * 16, 16)] * sv)
                return 0
            lax.fori_loop(0, _CAP // 16, _scale, 0)
            pltpu.sync_copy(grows, acc.at[floc], add=True)
            for q in range(4):
                ccol[pl.ds(q * 16, 16)] = ccol[pl.ds(_CAP + q * 16, 16)]
                cloc[pl.ds(q * 16, 16)] = cloc[pl.ds(_CAP + q * 16, 16)]
                cval[pl.ds(q * 16, 16)] = cval[pl.ds(_CAP + q * 16, 16)]

        def chunk_body(ch, _):
            r0 = ch * _C
            for j in range(6):
                pltpu.sync_copy(z_h, acc.at[pl.ds(s * _CPT + j * 128, 128)])
            pltpu.sync_copy(z_h.at[pl.ds(0, 16)],
                            acc.at[pl.ds(s * _CPT + 768, 16)])
            plsc.subcore_barrier()

            def issue(b, p):
                eb = b * _BS
                pltpu.async_copy(rows_h.at[pl.ds(base_e + eb, _BS)],
                                 rblk.at[p], sem)
                pltpu.async_copy(cols_h.at[pl.ds(base_e + eb, _BS)],
                                 cblk.at[p], sem)
                pltpu.async_copy(vals_h.at[pl.ds(base_e + eb, _BS)],
                                 vblk.at[p], sem)

            def drain3(b, p):
                eb = b * _BS
                for ref in (rblk, cblk, vblk):
                    pltpu.make_async_copy(
                        rows_h.at[pl.ds(base_e + eb, _BS)],
                        ref.at[p], sem).wait()

            def block_body(b, off):
                p = lax.rem(b, 2)
                drain3(b, p)
                pl.when(b + 1 < NB)(lambda: issue(b + 1, 1 - p))

                def group4(g4, off):
                    for kk in range(4):
                        gb = g4 * 64 + kk * 16
                        r = rblk[p, pl.ds(gb, 16)]
                        cc = cblk[p, pl.ds(gb, 16)]
                        vv = vblk[p, pl.ds(gb, 16)]
                        m = jnp.logical_and(r >= r0, r < r0 + _C)
                        loc = jnp.where(m, r - r0, _C)
                        pcs = plsc.cumsum(jnp.where(m, 1, 0))
                        dst = jnp.where(m, off + pcs - 1, _WIN)
                        plsc.store_scatter(ccol, [dst], cc)
                        plsc.store_scatter(cloc, [dst], loc)
                        plsc.store_scatter(cval, [dst], vv)
                        off = off + jnp.max(pcs)
                    do_flush = off >= _CAP
                    pl.when(do_flush)(flush)
                    return jnp.where(do_flush, off - _CAP, off)

                return lax.fori_loop(0, _BS // 64, group4, off)

            issue(0, 0)
            off = lax.fori_loop(0, NB, block_body, jnp.int32(0))

            def tail():
                for q in range(_COMP // 16):
                    idx = lane + q * 16
                    keep = idx < off
                    lq = cloc[pl.ds(q * 16, 16)]
                    vq = cval[pl.ds(q * 16, 16)]
                    cloc[pl.ds(q * 16, 16)] = jnp.where(keep, lq, _C)
                    cval[pl.ds(q * 16, 16)] = jnp.where(keep, vq, 0.0)
                flush()
            pl.when(off > 0)(tail)

            plsc.subcore_barrier()
            pltpu.sync_copy(
                acc.at[pl.ds(s * _CPT, _CPT)],
                out_h.at[c].at[pl.ds(r0 + s * _CPT, _CPT)])
            plsc.subcore_barrier()
            return 0

        lax.fori_loop(0, chunks, chunk_body, 0)

    return k(rows, cols, vals, Y, jnp.zeros((128, _D), jnp.float32))


_GB = 128


def _gather_product_sc(X0, idx_list, n_rows):
    nf = len(idx_list)
    per_w = ((n_rows + _NW * _GB - 1) // (_NW * _GB)) * _GB
    n_pad = per_w * _NW
    idxs = []
    for ix in idx_list:
        ix = ix.astype(jnp.int32)
        if n_pad != n_rows:
            ix = jnp.concatenate(
                [ix, jnp.zeros((n_pad - n_rows,), jnp.int32)])
        idxs.append(ix)
    nblk = per_w // _GB

    mesh = plsc.VectorSubcoreMesh(core_axis_name="c", subcore_axis_name="s")
    scratch = ([pltpu.VMEM((2, _GB), jnp.int32) for _ in range(nf)]
               + [pltpu.VMEM((2, nf, _GB, _D), jnp.float32),
                  pltpu.SemaphoreType.DMA])

    @functools.partial(
        pl.kernel,
        mesh=mesh,
        compiler_params=pltpu.CompilerParams(needs_layout_passes=False),
        out_type=jax.ShapeDtypeStruct((n_pad, _D), jnp.float32),
        scratch_types=scratch,
    )
    def k(x_h, *refs):
        idx_h = refs[:nf]
        out_h = refs[nf]
        ib = refs[nf + 1:nf + 1 + nf]
        rb = refs[nf + 1 + nf]
        sem = refs[nf + 2 + nf]
        c = lax.axis_index("c")
        s = lax.axis_index("s")
        wid = s * _NC + c
        base = wid * per_w

        def issue_rows(b, p):
            return [pltpu.async_copy(x_h.at[ib[f].at[p]], rb.at[p, f], sem)
                    for f in range(nf)]

        o0 = base
        for f in range(nf):
            pltpu.sync_copy(idx_h[f].at[pl.ds(o0, _GB)], ib[f].at[0])
        ds_cur = issue_rows(0, 0)

        for b in range(nblk):
            p = b % 2
            for d in ds_cur:
                d.wait()
            if b + 1 < nblk:
                on = base + (b + 1) * _GB
                for f in range(nf):
                    pltpu.sync_copy(idx_h[f].at[pl.ds(on, _GB)], ib[f].at[1 - p])
                ds_cur = issue_rows(b + 1, 1 - p)

            def mrow(i, _):
                for g in range(8):
                    sl = pl.ds(g * 16, 16)
                    m = rb[p, 0, i, sl] != 0.0
                    for f in range(1, nf):
                        m = jnp.logical_and(m, rb[p, f, i, sl] != 0.0)
                    rb[p, 0, i, sl] = jnp.where(m, 1.0, 0.0)
                return 0
            lax.fori_loop(0, _GB, mrow, 0)
            pltpu.sync_copy(rb.at[p, 0], out_h.at[pl.ds(base + b * _GB, _GB)])

    return k(X0, *idxs)


def _mm_bin_body(x_ref, w_ref, b_ref, o_ref):
    xb = jnp.where(x_ref[...] != 0, 1.0, 0.0)
    o_ref[...] = (
        jnp.dot(xb, w_ref[...], preferred_element_type=jnp.float32)
        + b_ref[...])


def _mm_bin(X, W, b, block=512):
    N, K = X.shape
    F = W.shape[1]
    return pl.pallas_call(
        _mm_bin_body,
        grid=(pl.cdiv(N, block),),
        in_specs=[
            pl.BlockSpec((block, K), lambda i: (i, 0)),
            pl.BlockSpec((K, F), lambda i: (0, 0)),
            pl.BlockSpec((1, F), lambda i: (0, 0)),
        ],
        out_specs=pl.BlockSpec((block, F), lambda i: (i, 0)),
        out_shape=jax.ShapeDtypeStruct((N, F), jnp.float32),
    )(X, W, b.reshape(1, F))


def _mm3_body(x_ref, w_ref, b_ref, o_ref):
    o_ref[0] = (
        jnp.dot(x_ref[...], w_ref[0], preferred_element_type=jnp.float32)
        + b_ref[0])


def _mm3(X, W3, b3, block=512):
    N, K = X.shape
    return pl.pallas_call(
        _mm3_body,
        grid=(3, pl.cdiv(N, block)),
        in_specs=[
            pl.BlockSpec((block, K), lambda g, i: (i, 0)),
            pl.BlockSpec((1, K, _D), lambda g, i: (g, 0, 0)),
            pl.BlockSpec((1, 1, _D), lambda g, i: (g, 0, 0)),
        ],
        out_specs=pl.BlockSpec((1, block, _D), lambda g, i: (g, i, 0)),
        out_shape=jax.ShapeDtypeStruct((3, N, _D), jnp.float32),
    )(X, W3, b3.reshape(3, 1, _D))


def _prelu2_body(a_ref, b_ref, w_ref, o_ref):
    h = a_ref[...] + b_ref[...]
    o_ref[...] = jnp.where(h >= 0, h, w_ref[0, 0] * h)


def _prelu_sum2(a, b, w, block=1024):
    N, F = a.shape
    return pl.pallas_call(
        _prelu2_body,
        grid=(pl.cdiv(N, block),),
        in_specs=[
            pl.BlockSpec((block, F), lambda i: (i, 0)),
            pl.BlockSpec((block, F), lambda i: (i, 0)),
            pl.BlockSpec((1, 1), lambda i: (0, 0)),
        ],
        out_specs=pl.BlockSpec((block, F), lambda i: (i, 0)),
        out_shape=jax.ShapeDtypeStruct((N, F), jnp.float32),
    )(a, b, w.reshape(1, 1))


def _tri_body(t0_ref, t1_ref, h0_ref, h1_ref, w_ref, b_ref, pw_ref, o_ref):
    h = h0_ref[...] + h1_ref[...]
    x1h = jnp.where(h >= 0, h, pw_ref[0, 0] * h)
    t = t0_ref[...] + t1_ref[...]
    o_ref[...] = x1h + (
        jnp.dot(t, w_ref[...], preferred_element_type=jnp.float32)
        + b_ref[...])


def _tri_merge(T0, T1, H0, H1, W, b, pw, block=512):
    N = T0.shape[0]
    return pl.pallas_call(
        _tri_body,
        grid=(pl.cdiv(N, block),),
        in_specs=[pl.BlockSpec((block, _D), lambda i: (i, 0))] * 4 + [
            pl.BlockSpec((_D, _D), lambda i: (0, 0)),
            pl.BlockSpec((1, _D), lambda i: (0, 0)),
            pl.BlockSpec((1, 1), lambda i: (0, 0)),
        ],
        out_specs=pl.BlockSpec((block, _D), lambda i: (i, 0)),
        out_shape=jax.ShapeDtypeStruct((N, _D), jnp.float32),
    )(T0, T1, H0, H1, W, b.reshape(1, _D), pw.reshape(1, 1))


def _final_body(h0_ref, h1_ref, s0_ref, s1_ref, pw_ref, o_ref):
    h = h0_ref[...] + h1_ref[...]
    x0h = jnp.where(h >= 0, h, pw_ref[0, 0] * h)
    o_ref[...] = (x0h + s0_ref[...] + s1_ref[...]) / 3.0


def _final(H0, H1, S0, S1, pw, block=1024):
    N = H0.shape[0]
    return pl.pallas_call(
        _final_body,
        grid=(pl.cdiv(N, block),),
        in_specs=[pl.BlockSpec((block, _D), lambda i: (i, 0))] * 4 + [
            pl.BlockSpec((1, 1), lambda i: (0, 0)),
        ],
        out_specs=pl.BlockSpec((block, _D), lambda i: (i, 0)),
        out_shape=jax.ShapeDtypeStruct((N, _D), jnp.float32),
    )(H0, H1, S0, S1, pw.reshape(1, 1))


def kernel(X0, X1_idx, X2_idx, L0_rows, L0_cols, L0_vals, L1_rows, L1_cols, L1_vals, L1u_rows, L1u_cols, L1u_vals, L1d_rows, L1d_cols, L1d_vals, L2_rows, L2_cols, L2_vals, B1_rows, B1_cols, B1_vals, B2_rows, B2_cols, B2_vals, Wn_u, bn_u, Wn_d, bn_d, Wn_p, bn_p, We_u, be_u, We_d, be_d, We_p, be_p, Wt_u, bt_u, Wt_d, bt_d, Wt_p, bt_p, W_tri, b_tri, prelu_w):
    N0 = X0.shape[0]
    N1 = X1_idx.shape[0]
    N2 = X2_idx.shape[0]

    Y0 = _mm_bin(X0, Wn_p + Wn_d, bn_p + bn_d)
    P0 = _spmm_sc(L0_rows, L0_cols, L0_vals, Y0, N0)
    H0a, H0b = P0[0, :N0], P0[1, :N0]

    X1f = _gather_product_sc(X0, [X1_idx[:, 0], X1_idx[:, 1]], N1)[:N1]
    W3 = jnp.stack([We_p, We_u, We_d])
    b3 = jnp.stack([be_p, be_u, be_d])
    Y1 = _mm3(X1f, W3, b3).reshape(3 * N1, _D)
    r1 = jnp.concatenate([L1_rows, L1u_rows, L1d_rows]).astype(jnp.int32)
    c1 = jnp.concatenate(
        [L1_cols.astype(jnp.int32),
         L1u_cols.astype(jnp.int32) + N1,
         L1d_cols.astype(jnp.int32) + 2 * N1])
    v1 = jnp.concatenate([L1_vals, L1u_vals, L1d_vals])
    P1 = _spmm_sc(r1, c1, v1, Y1, N1)
    H1a, H1b = P1[0, :N1], P1[1, :N1]

    X2f = _gather_product_sc(
        X0, [X2_idx[:, 0], X2_idx[:, 1], X2_idx[:, 2]], N2)[:N2]
    Y2 = _mm_bin(X2f, Wt_p + Wt_u, bt_p + bt_u)
    P2 = _spmm_sc(L2_rows, L2_cols, L2_vals, Y2, N2)
    X2h = _prelu_sum2(P2[0, :N2], P2[1, :N2], prelu_w)

    PT = _spmm_sc(B2_rows, B2_cols, B2_vals, X2h, N1)
    Sin = _tri_merge(PT[0, :N1], PT[1, :N1], H1a, H1b, W_tri, b_tri, prelu_w)
    PS = _spmm_sc(B1_rows, B1_cols, B1_vals, Sin, N0)
    return _final(H0a, H0b, PS[0, :N0], PS[1, :N0], prelu_w)


# speedup vs baseline: 1.3472x; 1.2228x over previous
import functools

import jax
import jax.numpy as jnp
from jax import lax
from jax.experimental import pallas as pl
from jax.experimental.pallas import tpu as pltpu
from jax.experimental.pallas import tpu_sc as plsc

_NC = 2
_NS = 16
_NW = _NC * _NS
_D = 128

_C = 12544
_CPT = _C // _NS
_CAP = 128
_BS = 2048
_WIN = _CAP + 64
_COMP = _WIN + 16


def _spmm_sc(rows, cols, vals, Y, n_out):
    E = rows.shape[0]
    ep_unit = _NS * _BS
    Ep = ((E + ep_unit - 1) // ep_unit) * ep_unit
    if Ep != E:
        pad = Ep - E
        z = jnp.zeros((pad,), jnp.int32)
        rows = jnp.concatenate([rows.astype(jnp.int32), z])
        cols = jnp.concatenate([cols.astype(jnp.int32), z])
        vals = jnp.concatenate([vals, jnp.zeros((pad,), vals.dtype)])
    else:
        rows = rows.astype(jnp.int32)
        cols = cols.astype(jnp.int32)
    chunks = (n_out + _C - 1) // _C
    chunks_pad = 2 * ((chunks + 1) // 2)
    n_pad = chunks_pad * _C
    Et = Ep // _NS
    NB = Et // _BS

    mesh = plsc.VectorSubcoreMesh(core_axis_name="c", subcore_axis_name="s")

    @functools.partial(
        pl.kernel,
        mesh=mesh,
        compiler_params=pltpu.CompilerParams(needs_layout_passes=False),
        out_type=jax.ShapeDtypeStruct((n_pad, _D), jnp.float32),
        scratch_types=[
            pltpu.VMEM_SHARED((_C + 8, _D), jnp.float32),
            pltpu.VMEM((2, _BS), jnp.int32),
            pltpu.VMEM((2, _BS), jnp.int32),
            pltpu.VMEM((2, _BS), jnp.float32),
            pltpu.VMEM((_COMP,), jnp.int32),
            pltpu.VMEM((_COMP,), jnp.int32),
            pltpu.VMEM((_COMP,), jnp.float32),
            pltpu.VMEM((_CAP,), jnp.int32),
            pltpu.VMEM((_CAP,), jnp.int32),
            pltpu.VMEM((_CAP, _D), jnp.float32),
            pltpu.SemaphoreType.DMA,
        ],
    )
    def k(rows_h, cols_h, vals_h, y_h, z_h, out_h,
          acc, rblk, cblk, vblk, ccol, cloc, cval, fcol, floc, grows, sem):
        c = lax.axis_index("c")
        s = lax.axis_index("s")
        base_e = s * Et
        zero16 = jnp.zeros((16,), jnp.float32)
        zero16i = jnp.zeros((16,), jnp.int32)
        lane = lax.iota(jnp.int32, 16)

        for q in range(_COMP // 16):
            ccol[pl.ds(q * 16, 16)] = zero16i
            cloc[pl.ds(q * 16, 16)] = zero16i
            cval[pl.ds(q * 16, 16)] = zero16

        def flush():
            for q in range(_CAP // 16):
                fcol[pl.ds(q * 16, 16)] = ccol[pl.ds(q * 16, 16)]
                floc[pl.ds(q * 16, 16)] = cloc[pl.ds(q * 16, 16)]
            pltpu.sync_copy(y_h.at[fcol], grows)
            def _scale(q, _):
                vv = cval[pl.ds(q * 16, 16)]
                for l in range(16):
                    sv = jnp.take(vv, jnp.full((16,), l, jnp.int32))
                    r = q * 16 + l
                    for g in range(8):
                        grows[r, pl.ds(g * 16, 16)] = (
                            grows[r, pl.ds(g * 16, 16)] * sv)
                return 0
            lax.fori_loop(0, _CAP // 16, _scale, 0)
            pltpu.sync_copy(grows, acc.at[floc], add=True)
            for q in range(4):
                ccol[pl.ds(q * 16, 16)] = ccol[pl.ds(_CAP + q * 16, 16)]
                cloc[pl.ds(q * 16, 16)] = cloc[pl.ds(_CAP + q * 16, 16)]
                cval[pl.ds(q * 16, 16)] = cval[pl.ds(_CAP + q * 16, 16)]

        def chunk_body(cp, _):
            ch = 2 * cp + c
            r0 = ch * _C
            for j in range(6):
                pltpu.sync_copy(z_h, acc.at[pl.ds(s * _CPT + j * 128, 128)])
            pltpu.sync_copy(z_h.at[pl.ds(0, 16)],
                            acc.at[pl.ds(s * _CPT + 768, 16)])
            plsc.subcore_barrier()

            def issue(b, p):
                eb = b * _BS
                pltpu.async_copy(rows_h.at[pl.ds(base_e + eb, _BS)],
                                 rblk.at[p], sem)
                pltpu.async_copy(cols_h.at[pl.ds(base_e + eb, _BS)],
                                 cblk.at[p], sem)
                pltpu.async_copy(vals_h.at[pl.ds(base_e + eb, _BS)],
                                 vblk.at[p], sem)

            def drain3(b, p):
                eb = b * _BS
                for ref in (rblk, cblk, vblk):
                    pltpu.make_async_copy(
                        rows_h.at[pl.ds(base_e + eb, _BS)],
                        ref.at[p], sem).wait()

            def block_body(b, off):
                p = lax.rem(b, 2)
                drain3(b, p)
                pl.when(b + 1 < NB)(lambda: issue(b + 1, 1 - p))

                def group4(g4, off):
                    for kk in range(4):
                        gb = g4 * 64 + kk * 16
                        r = rblk[p, pl.ds(gb, 16)]
                        cc = cblk[p, pl.ds(gb, 16)]
                        vv = vblk[p, pl.ds(gb, 16)]
                        m = jnp.logical_and(r >= r0, r < r0 + _C)
                        loc = jnp.where(m, r - r0, _C)
                        pcs = plsc.cumsum(jnp.where(m, 1, 0))
                        dst = jnp.where(m, off + pcs - 1, _WIN)
                        plsc.store_scatter(ccol, [dst], cc)
                        plsc.store_scatter(cloc, [dst], loc)
                        plsc.store_scatter(cval, [dst], vv)
                        off = off + jnp.max(pcs)
                    do_flush = off >= _CAP
                    pl.when(do_flush)(flush)
                    return jnp.where(do_flush, off - _CAP, off)

                return lax.fori_loop(0, _BS // 64, group4, off)

            issue(0, 0)
            off = lax.fori_loop(0, NB, block_body, jnp.int32(0))

            def tail():
                for q in range(_COMP // 16):
                    idx = lane + q * 16
                    keep = idx < off
                    lq = cloc[pl.ds(q * 16, 16)]
                    vq = cval[pl.ds(q * 16, 16)]
                    cloc[pl.ds(q * 16, 16)] = jnp.where(keep, lq, _C)
                    cval[pl.ds(q * 16, 16)] = jnp.where(keep, vq, 0.0)
                flush()
            pl.when(off > 0)(tail)

            plsc.subcore_barrier()
            pltpu.sync_copy(
                acc.at[pl.ds(s * _CPT, _CPT)],
                out_h.at[pl.ds(r0 + s * _CPT, _CPT)])
            plsc.subcore_barrier()
            return 0

        lax.fori_loop(0, chunks_pad // 2, chunk_body, 0)

    return k(rows, cols, vals, Y, jnp.zeros((128, _D), jnp.float32))


_GB = 128


def _gather_product_sc(X0, idx_list, n_rows):
    nf = len(idx_list)
    per_w = ((n_rows + _NW * _GB - 1) // (_NW * _GB)) * _GB
    n_pad = per_w * _NW
    idxs = []
    for ix in idx_list:
        ix = ix.astype(jnp.int32)
        if n_pad != n_rows:
            ix = jnp.concatenate(
                [ix, jnp.zeros((n_pad - n_rows,), jnp.int32)])
        idxs.append(ix)
    nblk = per_w // _GB

    mesh = plsc.VectorSubcoreMesh(core_axis_name="c", subcore_axis_name="s")
    scratch = ([pltpu.VMEM((2, _GB), jnp.int32) for _ in range(nf)]
               + [pltpu.VMEM((2, nf, _GB, _D), jnp.float32),
                  pltpu.SemaphoreType.DMA])

    @functools.partial(
        pl.kernel,
        mesh=mesh,
        compiler_params=pltpu.CompilerParams(needs_layout_passes=False),
        out_type=jax.ShapeDtypeStruct((n_pad, _D), jnp.float32),
        scratch_types=scratch,
    )
    def k(x_h, *refs):
        idx_h = refs[:nf]
        out_h = refs[nf]
        ib = refs[nf + 1:nf + 1 + nf]
        rb = refs[nf + 1 + nf]
        sem = refs[nf + 2 + nf]
        c = lax.axis_index("c")
        s = lax.axis_index("s")
        wid = s * _NC + c
        base = wid * per_w

        def issue_rows(b, p):
            return [pltpu.async_copy(x_h.at[ib[f].at[p]], rb.at[p, f], sem)
                    for f in range(nf)]

        o0 = base
        for f in range(nf):
            pltpu.sync_copy(idx_h[f].at[pl.ds(o0, _GB)], ib[f].at[0])
        ds_cur = issue_rows(0, 0)

        for b in range(nblk):
            p = b % 2
            for d in ds_cur:
                d.wait()
            if b + 1 < nblk:
                on = base + (b + 1) * _GB
                for f in range(nf):
                    pltpu.sync_copy(idx_h[f].at[pl.ds(on, _GB)], ib[f].at[1 - p])
                ds_cur = issue_rows(b + 1, 1 - p)

            def mrow(i, _):
                for g in range(8):
                    sl = pl.ds(g * 16, 16)
                    m = rb[p, 0, i, sl] != 0.0
                    for f in range(1, nf):
                        m = jnp.logical_and(m, rb[p, f, i, sl] != 0.0)
                    rb[p, 0, i, sl] = jnp.where(m, 1.0, 0.0)
                return 0
            lax.fori_loop(0, _GB, mrow, 0)
            pltpu.sync_copy(rb.at[p, 0], out_h.at[pl.ds(base + b * _GB, _GB)])

    return k(X0, *idxs)


def _mm_bin_body(x_ref, w_ref, b_ref, o_ref):
    xb = jnp.where(x_ref[...] != 0, 1.0, 0.0)
    o_ref[...] = (
        jnp.dot(xb, w_ref[...], preferred_element_type=jnp.float32)
        + b_ref[...])


def _mm_bin(X, W, b, block=512):
    N, K = X.shape
    F = W.shape[1]
    return pl.pallas_call(
        _mm_bin_body,
        grid=(pl.cdiv(N, block),),
        in_specs=[
            pl.BlockSpec((block, K), lambda i: (i, 0)),
            pl.BlockSpec((K, F), lambda i: (0, 0)),
            pl.BlockSpec((1, F), lambda i: (0, 0)),
        ],
        out_specs=pl.BlockSpec((block, F), lambda i: (i, 0)),
        out_shape=jax.ShapeDtypeStruct((N, F), jnp.float32),
    )(X, W, b.reshape(1, F))


def _mm3_body(x_ref, w_ref, b_ref, o_ref):
    o_ref[0] = (
        jnp.dot(x_ref[...], w_ref[0], preferred_element_type=jnp.float32)
        + b_ref[0])


def _mm3(X, W3, b3, block=512):
    N, K = X.shape
    return pl.pallas_call(
        _mm3_body,
        grid=(3, pl.cdiv(N, block)),
        in_specs=[
            pl.BlockSpec((block, K), lambda g, i: (i, 0)),
            pl.BlockSpec((1, K, _D), lambda g, i: (g, 0, 0)),
            pl.BlockSpec((1, 1, _D), lambda g, i: (g, 0, 0)),
        ],
        out_specs=pl.BlockSpec((1, block, _D), lambda g, i: (g, i, 0)),
        out_shape=jax.ShapeDtypeStruct((3, N, _D), jnp.float32),
    )(X, W3, b3.reshape(3, 1, _D))


def _prelu1_body(a_ref, w_ref, o_ref):
    h = a_ref[...]
    o_ref[...] = jnp.where(h >= 0, h, w_ref[0, 0] * h)


def _prelu(a, w, block=1024):
    N, F = a.shape
    return pl.pallas_call(
        _prelu1_body,
        grid=(pl.cdiv(N, block),),
        in_specs=[
            pl.BlockSpec((block, F), lambda i: (i, 0)),
            pl.BlockSpec((1, 1), lambda i: (0, 0)),
        ],
        out_specs=pl.BlockSpec((block, F), lambda i: (i, 0)),
        out_shape=jax.ShapeDtypeStruct((N, F), jnp.float32),
    )(a, w.reshape(1, 1))


def _tri_body(t_ref, h_ref, w_ref, b_ref, pw_ref, o_ref):
    h = h_ref[...]
    x1h = jnp.where(h >= 0, h, pw_ref[0, 0] * h)
    o_ref[...] = x1h + (
        jnp.dot(t_ref[...], w_ref[...], preferred_element_type=jnp.float32)
        + b_ref[...])


def _tri_merge(T, H, W, b, pw, block=512):
    N = T.shape[0]
    return pl.pallas_call(
        _tri_body,
        grid=(pl.cdiv(N, block),),
        in_specs=[pl.BlockSpec((block, _D), lambda i: (i, 0))] * 2 + [
            pl.BlockSpec((_D, _D), lambda i: (0, 0)),
            pl.BlockSpec((1, _D), lambda i: (0, 0)),
            pl.BlockSpec((1, 1), lambda i: (0, 0)),
        ],
        out_specs=pl.BlockSpec((block, _D), lambda i: (i, 0)),
        out_shape=jax.ShapeDtypeStruct((N, _D), jnp.float32),
    )(T, H, W, b.reshape(1, _D), pw.reshape(1, 1))


def _final_body(h_ref, s_ref, pw_ref, o_ref):
    h = h_ref[...]
    x0h = jnp.where(h >= 0, h, pw_ref[0, 0] * h)
    o_ref[...] = (x0h + s_ref[...]) / 3.0


def _final(H, S, pw, block=1024):
    N = H.shape[0]
    return pl.pallas_call(
        _final_body,
        grid=(pl.cdiv(N, block),),
        in_specs=[pl.BlockSpec((block, _D), lambda i: (i, 0))] * 2 + [
            pl.BlockSpec((1, 1), lambda i: (0, 0)),
        ],
        out_specs=pl.BlockSpec((block, _D), lambda i: (i, 0)),
        out_shape=jax.ShapeDtypeStruct((N, _D), jnp.float32),
    )(H, S, pw.reshape(1, 1))


def kernel(X0, X1_idx, X2_idx, L0_rows, L0_cols, L0_vals, L1_rows, L1_cols, L1_vals, L1u_rows, L1u_cols, L1u_vals, L1d_rows, L1d_cols, L1d_vals, L2_rows, L2_cols, L2_vals, B1_rows, B1_cols, B1_vals, B2_rows, B2_cols, B2_vals, Wn_u, bn_u, Wn_d, bn_d, Wn_p, bn_p, We_u, be_u, We_d, be_d, We_p, be_p, Wt_u, bt_u, Wt_d, bt_d, Wt_p, bt_p, W_tri, b_tri, prelu_w):
    N0 = X0.shape[0]
    N1 = X1_idx.shape[0]
    N2 = X2_idx.shape[0]

    Y0 = _mm_bin(X0, Wn_p + Wn_d, bn_p + bn_d)
    H0 = _spmm_sc(L0_rows, L0_cols, L0_vals, Y0, N0)[:N0]

    X1f = _gather_product_sc(X0, [X1_idx[:, 0], X1_idx[:, 1]], N1)[:N1]
    W3 = jnp.stack([We_p, We_u, We_d])
    b3 = jnp.stack([be_p, be_u, be_d])
    Y1 = _mm3(X1f, W3, b3).reshape(3 * N1, _D)
    r1 = jnp.concatenate([L1_rows, L1u_rows, L1d_rows]).astype(jnp.int32)
    c1 = jnp.concatenate(
        [L1_cols.astype(jnp.int32),
         L1u_cols.astype(jnp.int32) + N1,
         L1d_cols.astype(jnp.int32) + 2 * N1])
    v1 = jnp.concatenate([L1_vals, L1u_vals, L1d_vals])
    H1 = _spmm_sc(r1, c1, v1, Y1, N1)[:N1]

    X2f = _gather_product_sc(
        X0, [X2_idx[:, 0], X2_idx[:, 1], X2_idx[:, 2]], N2)[:N2]
    Y2 = _mm_bin(X2f, Wt_p + Wt_u, bt_p + bt_u)
    X2h = _prelu(_spmm_sc(L2_rows, L2_cols, L2_vals, Y2, N2)[:N2], prelu_w)

    T = _spmm_sc(B2_rows, B2_cols, B2_vals, X2h, N1)[:N1]
    Sin = _tri_merge(T, H1, W_tri, b_tri, prelu_w)
    S = _spmm_sc(B1_rows, B1_cols, B1_vals, Sin, N0)[:N0]
    return _final(H0, S, prelu_w)


# speedup vs baseline: 1.4220x; 1.0555x over previous
import functools

import jax
import jax.numpy as jnp
from jax import lax
from jax.experimental import pallas as pl
from jax.experimental.pallas import tpu as pltpu
from jax.experimental.pallas import tpu_sc as plsc

_NC = 2
_NS = 16
_NW = _NC * _NS
_D = 128

_C = 12544
_CPT = _C // _NS
_CAP = 128
_BS = 2048
_WIN = _CAP + 64
_COMP = _WIN + 16


def _spmm_sc(rows, cols, vals, Y, n_out):
    E = rows.shape[0]
    ep_unit = _NS * _BS
    Ep = ((E + ep_unit - 1) // ep_unit) * ep_unit
    if Ep != E:
        pad = Ep - E
        z = jnp.zeros((pad,), jnp.int32)
        rows = jnp.concatenate([rows.astype(jnp.int32), z])
        cols = jnp.concatenate([cols.astype(jnp.int32), z])
        vals = jnp.concatenate([vals, jnp.zeros((pad,), vals.dtype)])
    else:
        rows = rows.astype(jnp.int32)
        cols = cols.astype(jnp.int32)
    chunks = (n_out + _C - 1) // _C
    chunks_pad = 2 * ((chunks + 1) // 2)
    n_pad = chunks_pad * _C
    Et = Ep // _NS
    NB = Et // _BS

    mesh = plsc.VectorSubcoreMesh(core_axis_name="c", subcore_axis_name="s")

    @functools.partial(
        pl.kernel,
        mesh=mesh,
        compiler_params=pltpu.CompilerParams(needs_layout_passes=False),
        out_type=jax.ShapeDtypeStruct((n_pad, _D), jnp.float32),
        scratch_types=[
            pltpu.VMEM_SHARED((_C + 8, _D), jnp.float32),
            pltpu.VMEM((2, _BS), jnp.int32),
            pltpu.VMEM((2, _BS), jnp.int32),
            pltpu.VMEM((2, _BS), jnp.float32),
            pltpu.VMEM((_COMP,), jnp.int32),
            pltpu.VMEM((_COMP,), jnp.int32),
            pltpu.VMEM((_COMP,), jnp.float32),
            pltpu.VMEM((_CAP,), jnp.int32),
            pltpu.VMEM((_CAP,), jnp.int32),
            pltpu.VMEM((_CAP, _D), jnp.float32),
            pltpu.SemaphoreType.DMA,
            pltpu.SemaphoreType.DMA,
            pltpu.SMEM((4,), jnp.int32),
        ],
    )
    def k(rows_h, cols_h, vals_h, y_h, z_h, out_h,
          acc, rblk, cblk, vblk, ccol, cloc, cval, fcol, floc, grows, sem,
          sem2, psm):
        c = lax.axis_index("c")
        s = lax.axis_index("s")
        base_e = s * Et
        zero16 = jnp.zeros((16,), jnp.float32)
        zero16i = jnp.zeros((16,), jnp.int32)
        lane = lax.iota(jnp.int32, 16)

        psm[0] = 0
        for q in range(_COMP // 16):
            ccol[pl.ds(q * 16, 16)] = zero16i
            cloc[pl.ds(q * 16, 16)] = zero16i
            cval[pl.ds(q * 16, 16)] = zero16

        def drain_scatter():
            pltpu.make_async_copy(grows, acc.at[floc], sem2).wait()

        def flush():
            pl.when(psm[0] == 1)(drain_scatter)
            psm[0] = 0
            for q in range(_CAP // 16):
                fcol[pl.ds(q * 16, 16)] = ccol[pl.ds(q * 16, 16)]
                floc[pl.ds(q * 16, 16)] = cloc[pl.ds(q * 16, 16)]
            pltpu.sync_copy(y_h.at[fcol], grows)
            def _scale(q, _):
                vv = cval[pl.ds(q * 16, 16)]
                for l in range(16):
                    sv = jnp.take(vv, jnp.full((16,), l, jnp.int32))
                    r = q * 16 + l
                    for g in range(8):
                        grows[r, pl.ds(g * 16, 16)] = (
                            grows[r, pl.ds(g * 16, 16)] * sv)
                return 0
            lax.fori_loop(0, _CAP // 16, _scale, 0)
            pltpu.async_copy(grows, acc.at[floc], sem2, add=True)
            psm[0] = 1
            for q in range(4):
                ccol[pl.ds(q * 16, 16)] = ccol[pl.ds(_CAP + q * 16, 16)]
                cloc[pl.ds(q * 16, 16)] = cloc[pl.ds(_CAP + q * 16, 16)]
                cval[pl.ds(q * 16, 16)] = cval[pl.ds(_CAP + q * 16, 16)]

        def chunk_body(cp, _):
            ch = 2 * cp + c
            r0 = ch * _C
            for j in range(6):
                pltpu.sync_copy(z_h, acc.at[pl.ds(s * _CPT + j * 128, 128)])
            pltpu.sync_copy(z_h.at[pl.ds(0, 16)],
                            acc.at[pl.ds(s * _CPT + 768, 16)])
            plsc.subcore_barrier()

            def issue(b, p):
                eb = b * _BS
                pltpu.async_copy(rows_h.at[pl.ds(base_e + eb, _BS)],
                                 rblk.at[p], sem)
                pltpu.async_copy(cols_h.at[pl.ds(base_e + eb, _BS)],
                                 cblk.at[p], sem)
                pltpu.async_copy(vals_h.at[pl.ds(base_e + eb, _BS)],
                                 vblk.at[p], sem)

            def drain3(b, p):
                eb = b * _BS
                for ref in (rblk, cblk, vblk):
                    pltpu.make_async_copy(
                        rows_h.at[pl.ds(base_e + eb, _BS)],
                        ref.at[p], sem).wait()

            def block_body(b, off):
                p = lax.rem(b, 2)
                drain3(b, p)
                pl.when(b + 1 < NB)(lambda: issue(b + 1, 1 - p))

                def group4(g4, off):
                    for kk in range(4):
                        gb = g4 * 64 + kk * 16
                        r = rblk[p, pl.ds(gb, 16)]
                        cc = cblk[p, pl.ds(gb, 16)]
                        vv = vblk[p, pl.ds(gb, 16)]
                        m = jnp.logical_and(r >= r0, r < r0 + _C)
                        loc = jnp.where(m, r - r0, _C)
                        pcs = plsc.cumsum(jnp.where(m, 1, 0))
                        dst = jnp.where(m, off + pcs - 1, _WIN)
                        plsc.store_scatter(ccol, [dst], cc)
                        plsc.store_scatter(cloc, [dst], loc)
                        plsc.store_scatter(cval, [dst], vv)
                        off = off + jnp.max(pcs)
                    do_flush = off >= _CAP
                    pl.when(do_flush)(flush)
                    return jnp.where(do_flush, off - _CAP, off)

                return lax.fori_loop(0, _BS // 64, group4, off)

            issue(0, 0)
            off = lax.fori_loop(0, NB, block_body, jnp.int32(0))

            def tail():
                for q in range(_COMP // 16):
                    idx = lane + q * 16
                    keep = idx < off
                    lq = cloc[pl.ds(q * 16, 16)]
                    vq = cval[pl.ds(q * 16, 16)]
                    cloc[pl.ds(q * 16, 16)] = jnp.where(keep, lq, _C)
                    cval[pl.ds(q * 16, 16)] = jnp.where(keep, vq, 0.0)
                flush()
            pl.when(off > 0)(tail)
            pl.when(psm[0] == 1)(drain_scatter)
            psm[0] = 0

            plsc.subcore_barrier()
            pltpu.sync_copy(
                acc.at[pl.ds(s * _CPT, _CPT)],
                out_h.at[pl.ds(r0 + s * _CPT, _CPT)])
            plsc.subcore_barrier()
            return 0

        lax.fori_loop(0, chunks_pad // 2, chunk_body, 0)

    return k(rows, cols, vals, Y, jnp.zeros((128, _D), jnp.float32))


_GB = 128


def _gather_product_sc(X0, idx_list, n_rows):
    nf = len(idx_list)
    per_w = ((n_rows + _NW * _GB - 1) // (_NW * _GB)) * _GB
    n_pad = per_w * _NW
    idxs = []
    for ix in idx_list:
        ix = ix.astype(jnp.int32)
        if n_pad != n_rows:
            ix = jnp.concatenate(
                [ix, jnp.zeros((n_pad - n_rows,), jnp.int32)])
        idxs.append(ix)
    nblk = per_w // _GB

    mesh = plsc.VectorSubcoreMesh(core_axis_name="c", subcore_axis_name="s")
    scratch = ([pltpu.VMEM((2, _GB), jnp.int32) for _ in range(nf)]
               + [pltpu.VMEM((2, nf, _GB, _D), jnp.float32),
                  pltpu.SemaphoreType.DMA, pltpu.SemaphoreType.DMA])

    @functools.partial(
        pl.kernel,
        mesh=mesh,
        compiler_params=pltpu.CompilerParams(needs_layout_passes=False),
        out_type=jax.ShapeDtypeStruct((n_pad, _D), jnp.float32),
        scratch_types=scratch,
    )
    def k(x_h, *refs):
        idx_h = refs[:nf]
        out_h = refs[nf]
        ib = refs[nf + 1:nf + 1 + nf]
        rb = refs[nf + 1 + nf]
        sem = refs[nf + 2 + nf]
        sem_o = refs[nf + 3 + nf]
        c = lax.axis_index("c")
        s = lax.axis_index("s")
        wid = s * _NC + c
        base = wid * per_w

        def issue_rows(b, p):
            return [pltpu.async_copy(x_h.at[ib[f].at[p]], rb.at[p, f], sem)
                    for f in range(nf)]

        o0 = base
        for f in range(nf):
            pltpu.sync_copy(idx_h[f].at[pl.ds(o0, _GB)], ib[f].at[0])
        ds_cur = issue_rows(0, 0)

        ds_out = [None, None]
        for b in range(nblk):
            p = b % 2
            for d in ds_cur:
                d.wait()
            if b + 1 < nblk:
                on = base + (b + 1) * _GB
                for f in range(nf):
                    pltpu.sync_copy(idx_h[f].at[pl.ds(on, _GB)], ib[f].at[1 - p])
                if ds_out[1 - p] is not None:
                    ds_out[1 - p].wait()
                    ds_out[1 - p] = None
                ds_cur = issue_rows(b + 1, 1 - p)
            if ds_out[p] is not None:
                ds_out[p].wait()
                ds_out[p] = None

            def mrow(i, _):
                for g in range(8):
                    sl = pl.ds(g * 16, 16)
                    m = rb[p, 0, i, sl] != 0.0
                    for f in range(1, nf):
                        m = jnp.logical_and(m, rb[p, f, i, sl] != 0.0)
                    rb[p, 0, i, sl] = jnp.where(m, 1.0, 0.0)
                return 0
            lax.fori_loop(0, _GB, mrow, 0)
            ds_out[p] = pltpu.async_copy(
                rb.at[p, 0], out_h.at[pl.ds(base + b * _GB, _GB)], sem_o)
        for d in ds_out:
            if d is not None:
                d.wait()

    return k(X0, *idxs)


def _mm_bin_body(x_ref, w_ref, b_ref, o_ref):
    xb = jnp.where(x_ref[...] != 0, 1.0, 0.0)
    o_ref[...] = (
        jnp.dot(xb, w_ref[...], preferred_element_type=jnp.float32)
        + b_ref[...])


def _mm_bin(X, W, b, block=512):
    N, K = X.shape
    F = W.shape[1]
    return pl.pallas_call(
        _mm_bin_body,
        grid=(pl.cdiv(N, block),),
        in_specs=[
            pl.BlockSpec((block, K), lambda i: (i, 0)),
            pl.BlockSpec((K, F), lambda i: (0, 0)),
            pl.BlockSpec((1, F), lambda i: (0, 0)),
        ],
        out_specs=pl.BlockSpec((block, F), lambda i: (i, 0)),
        out_shape=jax.ShapeDtypeStruct((N, F), jnp.float32),
    )(X, W, b.reshape(1, F))


def _mm3_body(x_ref, w_ref, b_ref, o_ref):
    o_ref[0] = (
        jnp.dot(x_ref[...], w_ref[0], preferred_element_type=jnp.float32)
        + b_ref[0])


def _mm3(X, W3, b3, block=512):
    N, K = X.shape
    return pl.pallas_call(
        _mm3_body,
        grid=(3, pl.cdiv(N, block)),
        in_specs=[
            pl.BlockSpec((block, K), lambda g, i: (i, 0)),
            pl.BlockSpec((1, K, _D), lambda g, i: (g, 0, 0)),
            pl.BlockSpec((1, 1, _D), lambda g, i: (g, 0, 0)),
        ],
        out_specs=pl.BlockSpec((1, block, _D), lambda g, i: (g, i, 0)),
        out_shape=jax.ShapeDtypeStruct((3, N, _D), jnp.float32),
    )(X, W3, b3.reshape(3, 1, _D))


def _prelu1_body(a_ref, w_ref, o_ref):
    h = a_ref[...]
    o_ref[...] = jnp.where(h >= 0, h, w_ref[0, 0] * h)


def _prelu(a, w, block=1024):
    N, F = a.shape
    return pl.pallas_call(
        _prelu1_body,
        grid=(pl.cdiv(N, block),),
        in_specs=[
            pl.BlockSpec((block, F), lambda i: (i, 0)),
            pl.BlockSpec((1, 1), lambda i: (0, 0)),
        ],
        out_specs=pl.BlockSpec((block, F), lambda i: (i, 0)),
        out_shape=jax.ShapeDtypeStruct((N, F), jnp.float32),
    )(a, w.reshape(1, 1))


def _tri_body(t_ref, h_ref, w_ref, b_ref, pw_ref, o_ref):
    h = h_ref[...]
    x1h = jnp.where(h >= 0, h, pw_ref[0, 0] * h)
    o_ref[...] = x1h + (
        jnp.dot(t_ref[...], w_ref[...], preferred_element_type=jnp.float32)
        + b_ref[...])


def _tri_merge(T, H, W, b, pw, block=512):
    N = T.shape[0]
    return pl.pallas_call(
        _tri_body,
        grid=(pl.cdiv(N, block),),
        in_specs=[pl.BlockSpec((block, _D), lambda i: (i, 0))] * 2 + [
            pl.BlockSpec((_D, _D), lambda i: (0, 0)),
            pl.BlockSpec((1, _D), lambda i: (0, 0)),
            pl.BlockSpec((1, 1), lambda i: (0, 0)),
        ],
        out_specs=pl.BlockSpec((block, _D), lambda i: (i, 0)),
        out_shape=jax.ShapeDtypeStruct((N, _D), jnp.float32),
    )(T, H, W, b.reshape(1, _D), pw.reshape(1, 1))


def _final_body(h_ref, s_ref, pw_ref, o_ref):
    h = h_ref[...]
    x0h = jnp.where(h >= 0, h, pw_ref[0, 0] * h)
    o_ref[...] = (x0h + s_ref[...]) / 3.0


def _final(H, S, pw, block=1024):
    N = H.shape[0]
    return pl.pallas_call(
        _final_body,
        grid=(pl.cdiv(N, block),),
        in_specs=[pl.BlockSpec((block, _D), lambda i: (i, 0))] * 2 + [
            pl.BlockSpec((1, 1), lambda i: (0, 0)),
        ],
        out_specs=pl.BlockSpec((block, _D), lambda i: (i, 0)),
        out_shape=jax.ShapeDtypeStruct((N, _D), jnp.float32),
    )(H, S, pw.reshape(1, 1))


def kernel(X0, X1_idx, X2_idx, L0_rows, L0_cols, L0_vals, L1_rows, L1_cols, L1_vals, L1u_rows, L1u_cols, L1u_vals, L1d_rows, L1d_cols, L1d_vals, L2_rows, L2_cols, L2_vals, B1_rows, B1_cols, B1_vals, B2_rows, B2_cols, B2_vals, Wn_u, bn_u, Wn_d, bn_d, Wn_p, bn_p, We_u, be_u, We_d, be_d, We_p, be_p, Wt_u, bt_u, Wt_d, bt_d, Wt_p, bt_p, W_tri, b_tri, prelu_w):
    N0 = X0.shape[0]
    N1 = X1_idx.shape[0]
    N2 = X2_idx.shape[0]

    Y0 = _mm_bin(X0, Wn_p + Wn_d, bn_p + bn_d)
    H0 = _spmm_sc(L0_rows, L0_cols, L0_vals, Y0, N0)[:N0]

    X1f = _gather_product_sc(X0, [X1_idx[:, 0], X1_idx[:, 1]], N1)[:N1]
    W3 = jnp.stack([We_p, We_u, We_d])
    b3 = jnp.stack([be_p, be_u, be_d])
    Y1 = _mm3(X1f, W3, b3).reshape(3 * N1, _D)
    r1 = jnp.concatenate([L1_rows, L1u_rows, L1d_rows]).astype(jnp.int32)
    c1 = jnp.concatenate(
        [L1_cols.astype(jnp.int32),
         L1u_cols.astype(jnp.int32) + N1,
         L1d_cols.astype(jnp.int32) + 2 * N1])
    v1 = jnp.concatenate([L1_vals, L1u_vals, L1d_vals])
    H1 = _spmm_sc(r1, c1, v1, Y1, N1)[:N1]

    X2f = _gather_product_sc(
        X0, [X2_idx[:, 0], X2_idx[:, 1], X2_idx[:, 2]], N2)[:N2]
    Y2 = _mm_bin(X2f, Wt_p + Wt_u, bt_p + bt_u)
    X2h = _prelu(_spmm_sc(L2_rows, L2_cols, L2_vals, Y2, N2)[:N2], prelu_w)

    T = _spmm_sc(B2_rows, B2_cols, B2_vals, X2h, N1)[:N1]
    Sin = _tri_merge(T, H1, W_tri, b_tri, prelu_w)
    S = _spmm_sc(B1_rows, B1_cols, B1_vals, Sin, N0)[:N0]
    return _final(H0, S, prelu_w)
